# Initial kernel scaffold; baseline (speedup 1.0000x reference)
import jax
import jax.numpy as jnp
from jax.experimental import pallas as pl


def kernel(x, edge_index, W1, b1, g1, be1, W2, b2, g2, be2):
    raise NotImplementedError("write your pallas kernel here")


# speedup vs baseline: 11.2704x; 11.2704x over previous
import functools

import jax
import jax.numpy as jnp
from jax import lax
from jax.experimental import pallas as pl
from jax.experimental.pallas import tpu as pltpu
from jax.experimental.pallas import tpu_sc as plsc

N = 10000
E = 320000
D = 128
H = 128

NC = 2
NS = 16
NW = NC * NS

K = 128
EPT = 10112
NCHUNK = EPT // K
E_PAD = EPT * NW
N_PAD = 10112
RPT = N_PAD // NS

_mesh = plsc.VectorSubcoreMesh(core_axis_name="c", subcore_axis_name="s")


@functools.partial(
    pl.kernel,
    mesh=_mesh,
    out_type=jax.ShapeDtypeStruct((NC, N_PAD, 16), jnp.float32),
    scratch_types=[
        pltpu.VMEM((K,), jnp.int32),
        pltpu.VMEM((K, 16), jnp.float32),
        pltpu.VMEM_SHARED((N_PAD, 16), jnp.float32),
    ],
)
def _deg_pass(dst_hbm, ones_hbm, z_hbm, out_hbm, dst_v, ones_v, acc_sh):
    c = lax.axis_index("c")
    s = lax.axis_index("s")
    r0 = s * RPT
    pltpu.sync_copy(z_hbm.at[pl.ds(r0, RPT)], acc_sh.at[pl.ds(r0, RPT)])
    pltpu.sync_copy(ones_hbm, ones_v)
    plsc.subcore_barrier()
    base = (c * NS + s) * EPT

    @pl.loop(0, NCHUNK)
    def _(i):
        pltpu.sync_copy(dst_hbm.at[pl.ds(base + i * K, K)], dst_v)
        pltpu.sync_copy(ones_v, acc_sh.at[dst_v], add=True)

    plsc.subcore_barrier()
    pltpu.sync_copy(acc_sh.at[pl.ds(r0, RPT)], out_hbm.at[c, pl.ds(r0, RPT)])


@functools.partial(
    pl.kernel,
    mesh=_mesh,
    out_type=jax.ShapeDtypeStruct((NC, N_PAD, H), jnp.float32),
    scratch_types=[
        pltpu.VMEM((K,), jnp.int32),
        pltpu.VMEM((K,), jnp.int32),
        pltpu.VMEM((K, H), jnp.float32),
        pltpu.VMEM_SHARED((N_PAD, H), jnp.float32),
        pltpu.SemaphoreType.DMA,
    ],
)
def _edge_pass(hs_hbm, src_hbm, dst_hbm, z_hbm, out_hbm,
               src_v, dst_v, rows_v, acc_sh, sem):
    c = lax.axis_index("c")
    s = lax.axis_index("s")
    r0 = s * RPT
    pltpu.sync_copy(z_hbm.at[pl.ds(r0, RPT)], acc_sh.at[pl.ds(r0, RPT)])
    plsc.subcore_barrier()
    base = (c * NS + s) * EPT

    @pl.loop(0, NCHUNK)
    def _(i):
        off = base + i * K
        pltpu.sync_copy(src_hbm.at[pl.ds(off, K)], src_v)
        pltpu.sync_copy(dst_hbm.at[pl.ds(off, K)], dst_v)
        pltpu.async_copy(hs_hbm.at[src_v], rows_v, sem).wait()
        pltpu.sync_copy(rows_v, acc_sh.at[dst_v], add=True)

    plsc.subcore_barrier()
    pltpu.sync_copy(acc_sh.at[pl.ds(r0, RPT)], out_hbm.at[c, pl.ds(r0, RPT)])


def _prep_body(x_ref, w_ref, deg_ref, ht_ref, hs_ref, dinv_ref):
    deg = deg_ref[0, :, 0:1] + deg_ref[1, :, 0:1] + 1.0
    dinv = lax.rsqrt(deg)
    ht = jnp.dot(x_ref[...], w_ref[...], preferred_element_type=jnp.float32)
    ht_ref[...] = ht
    hs_ref[...] = ht * dinv
    dinv_ref[...] = dinv


def _mid_body(acc_ref, ht_ref, dinv_ref, b_ref, g_ref, be_ref, w2_ref,
              ht2_ref, hs2_ref):
    dinv = dinv_ref[...]
    u = (dinv * (acc_ref[0] + acc_ref[1])
         + (dinv * dinv) * ht_ref[...] + b_ref[...])
    uv = u[0:N, :]
    mean = jnp.mean(uv, axis=0, keepdims=True)
    var = jnp.mean((uv - mean) ** 2, axis=0, keepdims=True)
    y = jnp.maximum(
        g_ref[...] * (u - mean) * lax.rsqrt(var + 1e-5) + be_ref[...], 0.0)
    ht2 = jnp.dot(y, w2_ref[...], preferred_element_type=jnp.float32)
    ht2_ref[...] = ht2
    hs2_ref[...] = ht2 * dinv


def _fin_body(acc_ref, ht_ref, dinv_ref, b_ref, g_ref, be_ref, out_ref):
    dinv = dinv_ref[...]
    u = (dinv * (acc_ref[0] + acc_ref[1])
         + (dinv * dinv) * ht_ref[...] + b_ref[...])
    uv = u[0:N, :]
    mean = jnp.mean(uv, axis=0, keepdims=True)
    var = jnp.mean((uv - mean) ** 2, axis=0, keepdims=True)
    y = g_ref[...] * (uv - mean) * lax.rsqrt(var + 1e-5) + be_ref[...]
    out_ref[...] = jnp.maximum(y, 0.0)


def kernel(x, edge_index, W1, b1, g1, be1, W2, b2, g2, be2):
    f32 = jnp.float32
    src = edge_index[0]
    dst = edge_index[1]
    pad = jnp.full((E_PAD - E,), N, dtype=jnp.int32)
    src_p = jnp.concatenate([src, pad])
    dst_p = jnp.concatenate([dst, pad])
    x_p = jnp.pad(x, ((0, N_PAD - N), (0, 0)))
    ones16 = jnp.ones((K, 16), dtype=f32)
    zeros16 = jnp.zeros((N_PAD, 16), dtype=f32)
    zeros_h = jnp.zeros((N_PAD, H), dtype=f32)
    b1r, g1r, be1r = b1.reshape(1, H), g1.reshape(1, H), be1.reshape(1, H)
    b2r, g2r, be2r = b2.reshape(1, H), g2.reshape(1, H), be2.reshape(1, H)

    degacc = _deg_pass(dst_p, ones16, zeros16)

    ht1, hs1, dinv = pl.pallas_call(
        _prep_body,
        out_shape=[
            jax.ShapeDtypeStruct((N_PAD, H), f32),
            jax.ShapeDtypeStruct((N_PAD, H), f32),
            jax.ShapeDtypeStruct((N_PAD, 1), f32),
        ],
    )(x_p, W1, degacc)

    acc1 = _edge_pass(hs1, src_p, dst_p, zeros_h)

    ht2, hs2 = pl.pallas_call(
        _mid_body,
        out_shape=[
            jax.ShapeDtypeStruct((N_PAD, H), f32),
            jax.ShapeDtypeStruct((N_PAD, H), f32),
        ],
    )(acc1, ht1, dinv, b1r, g1r, be1r, W2)

    acc2 = _edge_pass(hs2, src_p, dst_p, zeros_h)

    out = pl.pallas_call(
        _fin_body,
        out_shape=jax.ShapeDtypeStruct((N, H), f32),
    )(acc2, ht2, dinv, b2r, g2r, be2r)
    return out
